# baseline (speedup 1.0000x reference)
import functools

import jax
import jax.numpy as jnp
from jax import lax
from jax.experimental import pallas as pl
from jax.experimental.pallas import tpu as pltpu
from jax.experimental.pallas import tpu_sc as plsc


def _sc_gather(table, idx):
    NT = idx.shape[0]
    D = table.shape[1]
    info = plsc.get_sparse_core_info()
    NC, NS = info.num_cores, info.num_subcores
    NW = NC * NS
    per_w = NT // NW
    CH = 128
    n_ch = per_w // CH

    mesh = plsc.VectorSubcoreMesh(core_axis_name="c", subcore_axis_name="s")

    @functools.partial(
        pl.kernel,
        mesh=mesh,
        out_type=jax.ShapeDtypeStruct((NT, D), jnp.float32),
        scratch_types=[
            pltpu.VMEM((CH,), jnp.int32),
            pltpu.VMEM((CH, D), jnp.float32),
            pltpu.SemaphoreType.DMA,
        ],
    )
    def gk(idx_hbm, table_hbm, out_hbm, idx_v, rows_v, sem):
        wid = lax.axis_index("s") * NC + lax.axis_index("c")
        base = wid * per_w
        for j in range(n_ch):
            off = base + j * CH
            pltpu.sync_copy(idx_hbm.at[pl.ds(off, CH)], idx_v)
            pltpu.async_copy(table_hbm.at[idx_v], rows_v, sem).wait()
            pltpu.sync_copy(rows_v, out_hbm.at[pl.ds(off, CH)])

    return gk(idx, table)


def _tc_body(w_ref, tt_ref, pw_ref, pb_ref, pos_ref, tb_ref, g_ref, bt_ref,
             o_ref):
    x = jnp.dot(w_ref[0], pw_ref[...], preferred_element_type=jnp.float32)
    x = x + pb_ref[...] + pos_ref[...]
    tid = tt_ref[0, 0].astype(jnp.float32)
    x = x + tb_ref[0:1, :] + tid * (tb_ref[1:2, :] - tb_ref[0:1, :])
    mean = jnp.mean(x, axis=1, keepdims=True)
    xc = x - mean
    var = jnp.mean(xc * xc, axis=1, keepdims=True)
    inv = lax.rsqrt(var + 1e-5)
    o_ref[0] = (xc * inv) * g_ref[...] + bt_ref[...]


def kernel(input_ids, token_type_ids, word_table, proj_W, proj_b,
           pos_table, type_table, ln_gamma, ln_beta):
    B, S = input_ids.shape
    V, E = word_table.shape
    H = proj_W.shape[1]
    BS = 256

    idx = input_ids.reshape(-1).astype(jnp.int32)
    gathered = _sc_gather(word_table, idx).reshape(B, S, E)
    tt4 = token_type_ids.astype(jnp.int32).reshape(B, S // BS, BS, 1)

    out = pl.pallas_call(
        _tc_body,
        grid=(S // BS, B),
        in_specs=[
            pl.BlockSpec((1, BS, E), lambda s, b: (b, s, 0)),
            pl.BlockSpec((1, 1, BS, 1), lambda s, b: (b, s, 0, 0)),
            pl.BlockSpec((E, H), lambda s, b: (0, 0)),
            pl.BlockSpec((1, H), lambda s, b: (0, 0)),
            pl.BlockSpec((BS, H), lambda s, b: (s, 0)),
            pl.BlockSpec((2, H), lambda s, b: (0, 0)),
            pl.BlockSpec((1, H), lambda s, b: (0, 0)),
            pl.BlockSpec((1, H), lambda s, b: (0, 0)),
        ],
        out_specs=pl.BlockSpec((1, BS, H), lambda s, b: (b, s, 0)),
        out_shape=jax.ShapeDtypeStruct((B, S, H), jnp.float32),
    )(gathered, tt4, proj_W, proj_b.reshape(1, H), pos_table,
      type_table, ln_gamma.reshape(1, H), ln_beta.reshape(1, H))

    return out


# speedup vs baseline: 1.2389x; 1.2389x over previous
import functools

import jax
import jax.numpy as jnp
from jax import lax
from jax.experimental import pallas as pl
from jax.experimental.pallas import tpu as pltpu
from jax.experimental.pallas import tpu_sc as plsc


def _sc_gather(table, idx):
    NT = idx.shape[0]
    D = table.shape[1]
    info = plsc.get_sparse_core_info()
    NC, NS = info.num_cores, info.num_subcores
    NW = NC * NS
    per_w = NT // NW
    CH = 128
    n_ch = per_w // CH

    mesh = plsc.VectorSubcoreMesh(core_axis_name="c", subcore_axis_name="s")

    @functools.partial(
        pl.kernel,
        mesh=mesh,
        out_type=jax.ShapeDtypeStruct((NT, D), jnp.float32),
        scratch_types=[
            pltpu.VMEM((CH,), jnp.int32),
            pltpu.VMEM((CH, D), jnp.float32),
            pltpu.SemaphoreType.DMA,
        ],
    )
    def gk(idx_hbm, table_hbm, out_hbm, idx_v, rows_v, sem):
        wid = lax.axis_index("s") * NC + lax.axis_index("c")
        base = wid * per_w
        for j in range(n_ch):
            off = base + j * CH
            pltpu.sync_copy(idx_hbm.at[pl.ds(off, CH)], idx_v)
            pltpu.async_copy(table_hbm.at[idx_v], rows_v, sem).wait()
            pltpu.sync_copy(rows_v, out_hbm.at[pl.ds(off, CH)])

    return gk(idx, table)


def _tc_body(w_ref, tt_ref, pw_ref, pb_ref, pos_ref, tb_ref, g_ref, bt_ref,
             o_ref):
    x = jnp.dot(w_ref[0], pw_ref[...], preferred_element_type=jnp.float32)
    x = x + pb_ref[...] + pos_ref[...]
    tid = tt_ref[0, 0].astype(jnp.float32)
    x = x + tb_ref[0:1, :] + tid * (tb_ref[1:2, :] - tb_ref[0:1, :])
    mean = jnp.mean(x, axis=1, keepdims=True)
    xc = x - mean
    var = jnp.mean(xc * xc, axis=1, keepdims=True)
    inv = lax.rsqrt(var + 1e-5)
    o_ref[0] = (xc * inv) * g_ref[...] + bt_ref[...]


def kernel(input_ids, token_type_ids, word_table, proj_W, proj_b,
           pos_table, type_table, ln_gamma, ln_beta):
    B, S = input_ids.shape
    V, E = word_table.shape
    H = proj_W.shape[1]
    BS = 1024

    idx = input_ids.reshape(-1).astype(jnp.int32)
    gathered = _sc_gather(word_table, idx).reshape(B, S, E)
    tt4 = token_type_ids.astype(jnp.int32).reshape(B, S // BS, BS, 1)

    out = pl.pallas_call(
        _tc_body,
        grid=(S // BS, B),
        in_specs=[
            pl.BlockSpec((1, BS, E), lambda s, b: (b, s, 0)),
            pl.BlockSpec((1, 1, BS, 1), lambda s, b: (b, s, 0, 0)),
            pl.BlockSpec((E, H), lambda s, b: (0, 0)),
            pl.BlockSpec((1, H), lambda s, b: (0, 0)),
            pl.BlockSpec((BS, H), lambda s, b: (s, 0)),
            pl.BlockSpec((2, H), lambda s, b: (0, 0)),
            pl.BlockSpec((1, H), lambda s, b: (0, 0)),
            pl.BlockSpec((1, H), lambda s, b: (0, 0)),
        ],
        out_specs=pl.BlockSpec((1, BS, H), lambda s, b: (b, s, 0)),
        out_shape=jax.ShapeDtypeStruct((B, S, H), jnp.float32),
    )(gathered, tt4, proj_W, proj_b.reshape(1, H), pos_table,
      type_table, ln_gamma.reshape(1, H), ln_beta.reshape(1, H))

    return out


# speedup vs baseline: 1.2844x; 1.0367x over previous
import functools

import jax
import jax.numpy as jnp
from jax import lax
from jax.experimental import pallas as pl
from jax.experimental.pallas import tpu as pltpu
from jax.experimental.pallas import tpu_sc as plsc


def _sc_gather(table, idx):
    NT = idx.shape[0]
    D = table.shape[1]
    info = plsc.get_sparse_core_info()
    NC, NS = info.num_cores, info.num_subcores
    NW = NC * NS
    per_w = NT // NW
    CH = 128
    n_ch = per_w // CH

    mesh = plsc.VectorSubcoreMesh(core_axis_name="c", subcore_axis_name="s")

    @functools.partial(
        pl.kernel,
        mesh=mesh,
        out_type=jax.ShapeDtypeStruct((NT, D), jnp.float32),
        scratch_types=[
            pltpu.VMEM((CH,), jnp.int32),
            pltpu.VMEM((CH, D), jnp.float32),
            pltpu.SemaphoreType.DMA,
        ],
    )
    def gk(idx_hbm, table_hbm, out_hbm, idx_v, rows_v, sem):
        wid = lax.axis_index("s") * NC + lax.axis_index("c")
        base = wid * per_w
        for j in range(n_ch):
            off = base + j * CH
            pltpu.sync_copy(idx_hbm.at[pl.ds(off, CH)], idx_v)
            pltpu.async_copy(table_hbm.at[idx_v], rows_v, sem).wait()
            pltpu.sync_copy(rows_v, out_hbm.at[pl.ds(off, CH)])

    return gk(idx, table)


def _tc_body(w_ref, tt_ref, pw_ref, pb_ref, pos_ref, tb_ref, g_ref, bt_ref,
             o_ref):
    x = jnp.dot(w_ref[0], pw_ref[...], preferred_element_type=jnp.float32)
    x = x + pb_ref[...] + pos_ref[...]
    tid = tt_ref[0, 0].astype(jnp.float32)
    x = x + tb_ref[0:1, :] + tid * (tb_ref[1:2, :] - tb_ref[0:1, :])
    mean = jnp.mean(x, axis=1, keepdims=True)
    xc = x - mean
    var = jnp.mean(xc * xc, axis=1, keepdims=True)
    inv = lax.rsqrt(var + 1e-5)
    o_ref[0] = (xc * inv) * g_ref[...] + bt_ref[...]


def kernel(input_ids, token_type_ids, word_table, proj_W, proj_b,
           pos_table, type_table, ln_gamma, ln_beta):
    B, S = input_ids.shape
    V, E = word_table.shape
    H = proj_W.shape[1]
    BS = 2048

    idx = input_ids.reshape(-1).astype(jnp.int32)
    gathered = _sc_gather(word_table, idx).reshape(B, S, E)
    tt4 = token_type_ids.astype(jnp.int32).reshape(B, S // BS, BS, 1)

    out = pl.pallas_call(
        _tc_body,
        grid=(S // BS, B),
        in_specs=[
            pl.BlockSpec((1, BS, E), lambda s, b: (b, s, 0)),
            pl.BlockSpec((1, 1, BS, 1), lambda s, b: (b, s, 0, 0)),
            pl.BlockSpec((E, H), lambda s, b: (0, 0)),
            pl.BlockSpec((1, H), lambda s, b: (0, 0)),
            pl.BlockSpec((BS, H), lambda s, b: (s, 0)),
            pl.BlockSpec((2, H), lambda s, b: (0, 0)),
            pl.BlockSpec((1, H), lambda s, b: (0, 0)),
            pl.BlockSpec((1, H), lambda s, b: (0, 0)),
        ],
        out_specs=pl.BlockSpec((1, BS, H), lambda s, b: (b, s, 0)),
        out_shape=jax.ShapeDtypeStruct((B, S, H), jnp.float32),
    )(gathered, tt4, proj_W, proj_b.reshape(1, H), pos_table,
      type_table, ln_gamma.reshape(1, H), ln_beta.reshape(1, H))

    return out
